# baseline (speedup 1.0000x reference)
import dataclasses
import functools

import jax
import jax.numpy as jnp
from jax.experimental import pallas as pl
from jax.experimental.pallas import tpu as pltpu
from jax.experimental.pallas import tpu_sc as plsc

_N_CHANNELS = 96
_N_REGIONS = 128
_BOUND = 2.5
_SCALE = _N_REGIONS / (2.0 * _BOUND)
_SHIFT = _BOUND * _SCALE

_LANES = 16
_W = 224
_SLAB = 224
_CHUNK_ROWS = 56
_NBUF = 2

_NC = 2
_NS = 16


def _sc_pwlu(x2, a_flat, d_flat):
    rows, cols = x2.shape
    n_workers = _NC * _NS
    rows_per_worker = rows // n_workers
    slabs_per_worker = rows_per_worker // _SLAB
    chunks_per_slab = _SLAB // _CHUNK_ROWS

    mesh = plsc.VectorSubcoreMesh(core_axis_name="c", subcore_axis_name="s")
    cp = pltpu.CompilerParams()
    if "needs_layout_passes" in pltpu.CompilerParams.__dataclass_fields__:
        cp = dataclasses.replace(cp, needs_layout_passes=False)
    if "use_tc_tiling_on_sc" in pltpu.CompilerParams.__dataclass_fields__:
        cp = dataclasses.replace(cp, use_tc_tiling_on_sc=True)

    @functools.partial(
        pl.kernel,
        out_type=jax.ShapeDtypeStruct((rows, cols), jnp.float32),
        mesh=mesh,
        compiler_params=cp,
        scratch_types=[
            pltpu.VMEM((_N_CHANNELS * _N_REGIONS,), jnp.float32),
            pltpu.VMEM((_N_CHANNELS * _N_REGIONS,), jnp.float32),
            pltpu.VMEM((_NBUF, _CHUNK_ROWS, _W), jnp.float32),
            pltpu.VMEM((_NBUF, _CHUNK_ROWS, _W), jnp.float32),
            pltpu.SemaphoreType.DMA,
            pltpu.SemaphoreType.DMA,
            pltpu.SemaphoreType.DMA,
        ],
    )
    def run(x_hbm, a_hbm, d_hbm, o_hbm, a_v, d_v, x_b, o_b, sem_t, sem_i, sem_o):
        pltpu.async_copy(a_hbm, a_v, sem_t).wait()
        pltpu.async_copy(d_hbm, d_v, sem_t).wait()

        cid = jax.lax.axis_index("c")
        sid = jax.lax.axis_index("s")
        wid = sid * _NC + cid
        row0 = wid * rows_per_worker
        slab0 = wid * slabs_per_worker
        c0 = jax.lax.rem(slab0, _N_CHANNELS)

        n_chunks = slabs_per_worker * chunks_per_slab

        def chunk_row(k):
            return row0 + k * _CHUNK_ROWS

        def start_in(k, buf):
            return pltpu.make_async_copy(
                x_hbm.at[pl.ds(chunk_row(k), _CHUNK_ROWS), :],
                x_b.at[buf],
                sem_i,
            )

        def start_out(k, buf):
            return pltpu.make_async_copy(
                o_b.at[buf],
                o_hbm.at[pl.ds(chunk_row(k), _CHUNK_ROWS), :],
                sem_o,
            )

        def compute(buf, coff):
            @plsc.parallel_loop(0, _CHUNK_ROWS, 1, unroll=2)
            def _(r):
                @plsc.parallel_loop(0, _W, _LANES, unroll=_W // _LANES)
                def _(j):
                    xv = x_b[buf, r, pl.ds(j, _LANES)]
                    xn = xv * _SCALE + _SHIFT
                    cl = jnp.clip(xn, 0.0, float(_N_REGIONS - 1))
                    f = cl.astype(jnp.int32) + coff
                    av = plsc.load_gather(a_v, [f])
                    dv = plsc.load_gather(d_v, [f])
                    o_b[buf, r, pl.ds(j, _LANES)] = av + xn * dv

        start_in(0, 0).start()
        start_in(1, 1).start()

        def coff_vec(c):
            return jnp.broadcast_to((c * _N_REGIONS).astype(jnp.int32), (_LANES,))

        @pl.loop(0, n_chunks, step=_NBUF, init_carry=c0)
        def _(k, c):
            for b in range(_NBUF):
                kk = k + b
                start_in(kk, b).wait()
                @pl.when(kk >= _NBUF)
                def _():
                    start_out(kk - _NBUF, b).wait()

                compute(b, coff_vec(c))
                start_out(kk, b).start()

                @pl.when(kk + _NBUF < n_chunks)
                def _():
                    start_in(kk + _NBUF, b).start()

                bump = jax.lax.rem(kk, chunks_per_slab) == (chunks_per_slab - 1)
                c = jnp.where(bump, c + 1, c)
                c = jnp.where(c >= _N_CHANNELS, c - _N_CHANNELS, c)
            return c

        for t in range(_NBUF):
            b = (n_chunks - _NBUF + t) % _NBUF
            start_out(n_chunks - _NBUF + t, b).wait()

    return run(x2, a_flat, d_flat)


def kernel(x, points):
    b, c, h, w = x.shape
    d = points[:, 1:] - points[:, :-1]
    r = jnp.arange(_N_REGIONS, dtype=jnp.float32)
    a = points[:, :-1] - r[None, :] * d
    a_flat = a.reshape(-1)
    d_flat = d.reshape(-1)
    x2 = x.reshape(b * c * h, w)
    out = _sc_pwlu(x2, a_flat, d_flat)
    return out.reshape(x.shape)


# speedup vs baseline: 1.7778x; 1.7778x over previous
import dataclasses
import functools

import jax
import jax.numpy as jnp
from jax.experimental import pallas as pl
from jax.experimental.pallas import tpu as pltpu
from jax.experimental.pallas import tpu_sc as plsc

_N_CHANNELS = 96
_N_REGIONS = 128
_BOUND = 2.5
_SCALE = _N_REGIONS / (2.0 * _BOUND)
_SHIFT = _BOUND * _SCALE

_LANES = 16
_W = 224
_SLAB = 224
_CHUNK_ROWS = 56
_NBUF = 2

_NC = 2
_NS = 16


def _sc_pwlu(x2, a_flat, d_flat):
    rows, cols = x2.shape
    n_workers = _NC * _NS
    rows_per_worker = rows // n_workers
    slabs_per_worker = rows_per_worker // _SLAB
    chunks_per_slab = _SLAB // _CHUNK_ROWS

    mesh = plsc.VectorSubcoreMesh(core_axis_name="c", subcore_axis_name="s")
    cp = pltpu.CompilerParams()
    if "needs_layout_passes" in pltpu.CompilerParams.__dataclass_fields__:
        cp = dataclasses.replace(cp, needs_layout_passes=False)
    if "use_tc_tiling_on_sc" in pltpu.CompilerParams.__dataclass_fields__:
        cp = dataclasses.replace(cp, use_tc_tiling_on_sc=True)

    @functools.partial(
        pl.kernel,
        out_type=jax.ShapeDtypeStruct((rows, cols), jnp.float32),
        mesh=mesh,
        compiler_params=cp,
        scratch_types=[
            pltpu.VMEM((_N_CHANNELS * _N_REGIONS,), jnp.float32),
            pltpu.VMEM((_N_CHANNELS * _N_REGIONS,), jnp.float32),
            pltpu.VMEM((_NBUF, _CHUNK_ROWS, _W), jnp.float32),
            pltpu.VMEM((_NBUF, _CHUNK_ROWS, _W), jnp.float32),
            pltpu.SemaphoreType.DMA,
            pltpu.SemaphoreType.DMA,
            pltpu.SemaphoreType.DMA,
        ],
    )
    def run(x_hbm, a_hbm, d_hbm, o_hbm, a_v, d_v, x_b, o_b, sem_t, sem_i, sem_o):
        pltpu.async_copy(a_hbm, a_v, sem_t).wait()
        pltpu.async_copy(d_hbm, d_v, sem_t).wait()

        cid = jax.lax.axis_index("c")
        sid = jax.lax.axis_index("s")
        wid = sid * _NC + cid
        row0 = wid * rows_per_worker
        slab0 = wid * slabs_per_worker
        c0 = jax.lax.rem(slab0, _N_CHANNELS)

        n_chunks = slabs_per_worker * chunks_per_slab

        def chunk_row(k):
            return row0 + k * _CHUNK_ROWS

        def start_in(k, buf):
            return pltpu.make_async_copy(
                x_hbm.at[pl.ds(chunk_row(k), _CHUNK_ROWS), :],
                x_b.at[buf],
                sem_i,
            )

        def start_out(k, buf):
            return pltpu.make_async_copy(
                o_b.at[buf],
                o_hbm.at[pl.ds(chunk_row(k), _CHUNK_ROWS), :],
                sem_o,
            )

        def compute(buf, coff):
            @plsc.parallel_loop(0, _CHUNK_ROWS, 1)
            def _(r):
                @plsc.parallel_loop(0, _W, _LANES, unroll=_W // _LANES)
                def _(j):
                    xv = x_b[buf, r, pl.ds(j, _LANES)]
                    xn = xv * _SCALE + _SHIFT
                    cl = jnp.clip(xn, 0.0, float(_N_REGIONS - 1))
                    f = cl.astype(jnp.int32) + coff
                    av = plsc.load_gather(a_v, [f])
                    dv = plsc.load_gather(d_v, [f])
                    o_b[buf, r, pl.ds(j, _LANES)] = av + xn * dv

        start_in(0, 0).start()
        start_in(1, 1).start()

        def coff_vec(c):
            return jnp.broadcast_to((c * _N_REGIONS).astype(jnp.int32), (_LANES,))

        @pl.loop(0, n_chunks, step=_NBUF, init_carry=c0)
        def _(k, c):
            for b in range(_NBUF):
                kk = k + b
                start_in(kk, b).wait()
                @pl.when(kk >= _NBUF)
                def _():
                    start_out(kk - _NBUF, b).wait()

                compute(b, coff_vec(c))
                start_out(kk, b).start()

                @pl.when(kk + _NBUF < n_chunks)
                def _():
                    start_in(kk + _NBUF, b).start()

                bump = jax.lax.rem(kk, chunks_per_slab) == (chunks_per_slab - 1)
                c = jnp.where(bump, c + 1, c)
                c = jnp.where(c >= _N_CHANNELS, c - _N_CHANNELS, c)
            return c

        for t in range(_NBUF):
            b = (n_chunks - _NBUF + t) % _NBUF
            start_out(n_chunks - _NBUF + t, b).wait()

    return run(x2, a_flat, d_flat)


def kernel(x, points):
    b, c, h, w = x.shape
    d = points[:, 1:] - points[:, :-1]
    r = jnp.arange(_N_REGIONS, dtype=jnp.float32)
    a = points[:, :-1] - r[None, :] * d
    a_flat = a.reshape(-1)
    d_flat = d.reshape(-1)
    x2 = x.reshape(b * c * h, w)
    out = _sc_pwlu(x2, a_flat, d_flat)
    return out.reshape(x.shape)


# speedup vs baseline: 1.7849x; 1.0040x over previous
import dataclasses
import functools

import jax
import jax.numpy as jnp
from jax.experimental import pallas as pl
from jax.experimental.pallas import tpu as pltpu
from jax.experimental.pallas import tpu_sc as plsc

_N_CHANNELS = 96
_N_REGIONS = 128
_BOUND = 2.5
_SCALE = _N_REGIONS / (2.0 * _BOUND)
_SHIFT = _BOUND * _SCALE

_LANES = 16
_W = 224
_SLAB = 224
_CHUNK_ROWS = 56
_NBUF = 3

_NC = 2
_NS = 16


def _sc_pwlu(x2, a_flat, d_flat):
    rows, cols = x2.shape
    n_workers = _NC * _NS
    rows_per_worker = rows // n_workers
    slabs_per_worker = rows_per_worker // _SLAB
    chunks_per_slab = _SLAB // _CHUNK_ROWS

    mesh = plsc.VectorSubcoreMesh(core_axis_name="c", subcore_axis_name="s")
    cp = pltpu.CompilerParams()
    if "needs_layout_passes" in pltpu.CompilerParams.__dataclass_fields__:
        cp = dataclasses.replace(cp, needs_layout_passes=False)
    if "use_tc_tiling_on_sc" in pltpu.CompilerParams.__dataclass_fields__:
        cp = dataclasses.replace(cp, use_tc_tiling_on_sc=True)

    @functools.partial(
        pl.kernel,
        out_type=jax.ShapeDtypeStruct((rows, cols), jnp.float32),
        mesh=mesh,
        compiler_params=cp,
        scratch_types=[
            pltpu.VMEM((_N_CHANNELS * _N_REGIONS,), jnp.float32),
            pltpu.VMEM((_N_CHANNELS * _N_REGIONS,), jnp.float32),
            pltpu.VMEM((_NBUF, _CHUNK_ROWS, _W), jnp.float32),
            pltpu.VMEM((_NBUF, _CHUNK_ROWS, _W), jnp.float32),
            pltpu.SemaphoreType.DMA,
            pltpu.SemaphoreType.DMA,
            pltpu.SemaphoreType.DMA,
        ],
    )
    def run(x_hbm, a_hbm, d_hbm, o_hbm, a_v, d_v, x_b, o_b, sem_t, sem_i, sem_o):
        pltpu.async_copy(a_hbm, a_v, sem_t).wait()
        pltpu.async_copy(d_hbm, d_v, sem_t).wait()

        cid = jax.lax.axis_index("c")
        sid = jax.lax.axis_index("s")
        wid = sid * _NC + cid
        row0 = wid * rows_per_worker
        slab0 = wid * slabs_per_worker
        c0 = jax.lax.rem(slab0, _N_CHANNELS)

        n_chunks = slabs_per_worker * chunks_per_slab

        def chunk_row(k):
            return row0 + k * _CHUNK_ROWS

        def start_in(k, buf):
            return pltpu.make_async_copy(
                x_hbm.at[pl.ds(chunk_row(k), _CHUNK_ROWS), :],
                x_b.at[buf],
                sem_i,
            )

        def start_out(k, buf):
            return pltpu.make_async_copy(
                o_b.at[buf],
                o_hbm.at[pl.ds(chunk_row(k), _CHUNK_ROWS), :],
                sem_o,
            )

        def compute(buf, coff):
            @plsc.parallel_loop(0, _CHUNK_ROWS, 1)
            def _(r):
                @plsc.parallel_loop(0, _W, _LANES, unroll=_W // _LANES)
                def _(j):
                    xv = x_b[buf, r, pl.ds(j, _LANES)]
                    xn = xv * _SCALE + _SHIFT
                    cl = jnp.clip(xn, 0.0, float(_N_REGIONS - 1))
                    f = cl.astype(jnp.int32) + coff
                    av = plsc.load_gather(a_v, [f])
                    dv = plsc.load_gather(d_v, [f])
                    o_b[buf, r, pl.ds(j, _LANES)] = av + xn * dv

        for p in range(_NBUF):
            start_in(p, p).start()

        def coff_vec(c):
            return jnp.broadcast_to((c * _N_REGIONS).astype(jnp.int32), (_LANES,))

        @pl.loop(0, n_chunks, step=_NBUF, init_carry=c0)
        def _(k, c):
            for b in range(_NBUF):
                kk = k + b
                start_in(kk, b).wait()
                @pl.when(kk >= _NBUF)
                def _():
                    start_out(kk - _NBUF, b).wait()

                compute(b, coff_vec(c))
                start_out(kk, b).start()

                @pl.when(kk + _NBUF < n_chunks)
                def _():
                    start_in(kk + _NBUF, b).start()

                bump = jax.lax.rem(kk, chunks_per_slab) == (chunks_per_slab - 1)
                c = jnp.where(bump, c + 1, c)
                c = jnp.where(c >= _N_CHANNELS, c - _N_CHANNELS, c)
            return c

        for t in range(_NBUF):
            b = (n_chunks - _NBUF + t) % _NBUF
            start_out(n_chunks - _NBUF + t, b).wait()

    return run(x2, a_flat, d_flat)


def kernel(x, points):
    b, c, h, w = x.shape
    d = points[:, 1:] - points[:, :-1]
    r = jnp.arange(_N_REGIONS, dtype=jnp.float32)
    a = points[:, :-1] - r[None, :] * d
    a_flat = a.reshape(-1)
    d_flat = d.reshape(-1)
    x2 = x.reshape(b * c * h, w)
    out = _sc_pwlu(x2, a_flat, d_flat)
    return out.reshape(x.shape)


# speedup vs baseline: 2.3056x; 1.2917x over previous
import dataclasses
import functools

import jax
import jax.numpy as jnp
from jax.experimental import pallas as pl
from jax.experimental.pallas import tpu as pltpu
from jax.experimental.pallas import tpu_sc as plsc

_N_CHANNELS = 96
_N_REGIONS = 128
_BOUND = 2.5
_SCALE = _N_REGIONS / (2.0 * _BOUND)
_SHIFT = _BOUND * _SCALE

_LANES = 16
_W = 224
_SLAB = 224
_CHUNK_ROWS = 56
_NBUF = 3

_NC = 2
_NS = 16


def _sc_pwlu(x2, a_flat, d_flat):
    rows, cols = x2.shape
    n_workers = _NC * _NS
    rows_per_worker = rows // n_workers
    slabs_per_worker = rows_per_worker // _SLAB
    chunks_per_slab = _SLAB // _CHUNK_ROWS

    mesh = plsc.VectorSubcoreMesh(core_axis_name="c", subcore_axis_name="s")
    cp = pltpu.CompilerParams()
    if "needs_layout_passes" in pltpu.CompilerParams.__dataclass_fields__:
        cp = dataclasses.replace(cp, needs_layout_passes=False)
    if "use_tc_tiling_on_sc" in pltpu.CompilerParams.__dataclass_fields__:
        cp = dataclasses.replace(cp, use_tc_tiling_on_sc=True)

    @functools.partial(
        pl.kernel,
        out_type=jax.ShapeDtypeStruct((rows, cols), jnp.float32),
        mesh=mesh,
        compiler_params=cp,
        scratch_types=[
            pltpu.VMEM((_N_CHANNELS * _N_REGIONS,), jnp.float32),
            pltpu.VMEM((_N_CHANNELS * _N_REGIONS,), jnp.float32),
            pltpu.VMEM((_NBUF, _CHUNK_ROWS, _W), jnp.float32),
            pltpu.VMEM((_NBUF, _CHUNK_ROWS, _W), jnp.float32),
            pltpu.SemaphoreType.DMA,
            pltpu.SemaphoreType.DMA,
            pltpu.SemaphoreType.DMA,
        ],
    )
    def run(x_hbm, a_hbm, d_hbm, o_hbm, a_v, d_v, x_b, o_b, sem_t, sem_i, sem_o):
        pltpu.async_copy(a_hbm, a_v, sem_t).wait()
        pltpu.async_copy(d_hbm, d_v, sem_t).wait()

        cid = jax.lax.axis_index("c")
        sid = jax.lax.axis_index("s")
        wid = sid * _NC + cid
        row0 = wid * rows_per_worker
        slab0 = wid * slabs_per_worker
        c0 = jax.lax.rem(slab0, _N_CHANNELS)

        n_chunks = slabs_per_worker * chunks_per_slab

        def chunk_row(k):
            return row0 + k * _CHUNK_ROWS

        def start_in(k, buf):
            return pltpu.make_async_copy(
                x_hbm.at[pl.ds(chunk_row(k), _CHUNK_ROWS), :],
                x_b.at[buf],
                sem_i,
            )

        def start_out(k, buf):
            return pltpu.make_async_copy(
                o_b.at[buf],
                o_hbm.at[pl.ds(chunk_row(k), _CHUNK_ROWS), :],
                sem_o,
            )

        def compute(buf, c):
            base = c * _N_REGIONS
            a_t = a_v.at[pl.ds(base, _N_REGIONS)]
            d_t = d_v.at[pl.ds(base, _N_REGIONS)]

            @plsc.parallel_loop(0, _CHUNK_ROWS, 1)
            def _(r):
                @plsc.parallel_loop(0, _W, _LANES, unroll=_W // _LANES)
                def _(j):
                    xv = x_b[buf, r, pl.ds(j, _LANES)]
                    xn = xv * _SCALE + _SHIFT
                    cl = jnp.clip(xn, 0.0, float(_N_REGIONS - 1))
                    f = cl.astype(jnp.int32)
                    av = plsc.load_gather(a_t, [f])
                    dv = plsc.load_gather(d_t, [f])
                    o_b[buf, r, pl.ds(j, _LANES)] = av + xn * dv

        for p in range(_NBUF):
            start_in(p, p).start()

        @pl.loop(0, n_chunks, step=_NBUF, init_carry=c0)
        def _(k, c):
            for b in range(_NBUF):
                kk = k + b
                start_in(kk, b).wait()
                @pl.when(kk >= _NBUF)
                def _():
                    start_out(kk - _NBUF, b).wait()

                compute(b, c)
                start_out(kk, b).start()

                @pl.when(kk + _NBUF < n_chunks)
                def _():
                    start_in(kk + _NBUF, b).start()

                bump = jax.lax.rem(kk, chunks_per_slab) == (chunks_per_slab - 1)
                c = jnp.where(bump, c + 1, c)
                c = jnp.where(c >= _N_CHANNELS, c - _N_CHANNELS, c)
            return c

        for t in range(_NBUF):
            b = (n_chunks - _NBUF + t) % _NBUF
            start_out(n_chunks - _NBUF + t, b).wait()

    return run(x2, a_flat, d_flat)


def kernel(x, points):
    b, c, h, w = x.shape
    d = points[:, 1:] - points[:, :-1]
    r = jnp.arange(_N_REGIONS, dtype=jnp.float32)
    a = points[:, :-1] - r[None, :] * d
    a_flat = a.reshape(-1)
    d_flat = d.reshape(-1)
    x2 = x.reshape(b * c * h, w)
    out = _sc_pwlu(x2, a_flat, d_flat)
    return out.reshape(x.shape)
